# baseline (speedup 1.0000x reference)
import functools

import jax
import jax.numpy as jnp
from jax import lax
from jax.experimental import pallas as pl
from jax.experimental.pallas import tpu as pltpu
from jax.experimental.pallas import tpu_sc as plsc

_EMBED = 32
_BATCH = 16384
_FIELDS = 26
_B = _BATCH * _FIELDS
_NC = 2
_NS = 16
_NW = _NC * _NS
_BPW = _B // _NW
_CHUNK = 1664
_NCHUNK = _BPW // _CHUNK

_mesh = plsc.VectorSubcoreMesh(core_axis_name="c", subcore_axis_name="s")


@functools.partial(
    pl.kernel,
    mesh=_mesh,
    out_type=jax.ShapeDtypeStruct((_B, _EMBED), jnp.float32),
    scratch_types=[
        pltpu.VMEM((_NCHUNK, _CHUNK), jnp.int32),
        pltpu.VMEM((2, _CHUNK, _EMBED), jnp.float32),
        pltpu.SemaphoreType.DMA,
        pltpu.SemaphoreType.DMA,
    ],
    compiler_params=pltpu.CompilerParams(use_tc_tiling_on_sc=False),
)
def _gather(idx_hbm, table_hbm, out_hbm, idx_v, rows_v, sem0, sem1):
    wid = lax.axis_index("s") * _NC + lax.axis_index("c")
    base = wid * _BPW
    sems = (sem0, sem1)
    pltpu.sync_copy(idx_hbm.at[pl.ds(wid * _NCHUNK, _NCHUNK)], idx_v)
    pltpu.async_copy(table_hbm.at[idx_v.at[0]], rows_v.at[0], sems[0])
    for i in range(_NCHUNK):
        if i + 1 < _NCHUNK:
            pltpu.async_copy(
                table_hbm.at[idx_v.at[i + 1]], rows_v.at[(i + 1) % 2],
                sems[(i + 1) % 2])
        pltpu.make_async_copy(
            table_hbm.at[idx_v.at[i]], rows_v.at[i % 2], sems[i % 2]).wait()
        pltpu.sync_copy(rows_v.at[i % 2],
                        out_hbm.at[pl.ds(base + i * _CHUNK, _CHUNK)])


_SLAB_W = 128 * _EMBED
_NSLAB = _B // 128
_SPW = _NSLAB // _NW
_GRP = 4
_GPW = _SPW // _GRP
_GRP_W = _GRP * _SLAB_W


@functools.partial(
    pl.kernel,
    mesh=_mesh,
    out_type=jax.ShapeDtypeStruct((_FIELDS, _EMBED, _BATCH), jnp.float32),
    scratch_types=[
        pltpu.VMEM((_GRP_W,), jnp.float32),
        pltpu.VMEM((_GRP_W,), jnp.float32),
        pltpu.VMEM((1, _EMBED, _GRP * 128), jnp.float32),
        pltpu.VMEM((1, _EMBED, _GRP * 128), jnp.float32),
        pltpu.SemaphoreType.DMA,
        pltpu.SemaphoreType.DMA,
        pltpu.SemaphoreType.DMA,
        pltpu.SemaphoreType.DMA,
    ],
    compiler_params=pltpu.CompilerParams(
        use_tc_tiling_on_sc=True, needs_layout_passes=False),
)
def _format_out(lin_hbm, out_hbm, in_v0, in_v1, tr_v0, tr_v1,
                isem0, isem1, osem0, osem1):
    wid = lax.axis_index("s") * _NC + lax.axis_index("c")
    s0 = wid * _SPW
    in_bufs = (in_v0, in_v1)
    tr_bufs = (tr_v0, tr_v1)
    isems = (isem0, isem1)
    osems = (osem0, osem1)

    def out_dst(g):
        s = s0 + g * _GRP
        return out_hbm.at[pl.ds(s // 128, 1), :,
                          pl.ds((s % 128) * 128, _GRP * 128)]

    pltpu.async_copy(lin_hbm.at[pl.ds(s0 * _SLAB_W, _GRP_W)],
                     in_v0, isems[0])
    pltpu.async_copy(lin_hbm.at[pl.ds((s0 + _GRP) * _SLAB_W, _GRP_W)],
                     in_v1, isems[1])
    iotas = [lax.iota(jnp.int32, 16) * _EMBED + seg * 16 * _EMBED
             for seg in range(8)]

    def body(i, carry):
        for b in range(2):
            g = 2 * i + b
            in_v = in_bufs[b]
            tr_v = tr_bufs[b]
            pltpu.make_async_copy(
                lin_hbm.at[pl.ds((s0 + g * _GRP) * _SLAB_W, _GRP_W)],
                in_v, isems[b]).wait()

            @pl.when(g >= 2)
            def _():
                pltpu.make_async_copy(tr_v, out_dst(g - 2), osems[b]).wait()

            for sl in range(_GRP):
                for e in range(_EMBED):
                    vs = [plsc.load_gather(
                              in_v, [iotas[seg] + (sl * _SLAB_W + e)])
                          for seg in range(8)]
                    for seg in range(8):
                        tr_v[0, e, sl * 128 + seg * 16:
                             sl * 128 + (seg + 1) * 16] = vs[seg]

            pltpu.async_copy(tr_v, out_dst(g), osems[b])

            @pl.when(g + 2 < _GPW)
            def _():
                pltpu.async_copy(
                    lin_hbm.at[pl.ds((s0 + (g + 2) * _GRP) * _SLAB_W,
                                     _GRP_W)],
                    in_v, isems[b])
        return carry

    lax.fori_loop(0, _GPW // 2, body, 0)
    pltpu.make_async_copy(tr_v0, out_dst(_GPW - 2), osems[0]).wait()
    pltpu.make_async_copy(tr_v1, out_dst(_GPW - 1), osems[1]).wait()


def kernel(x, table):
    idx = x.T.reshape(_B // _CHUNK, _CHUNK).astype(jnp.int32)
    lin = _gather(idx, table)
    out_t = _format_out(lin.reshape(-1))
    return jnp.transpose(out_t, (2, 0, 1))


# speedup vs baseline: 1.0246x; 1.0246x over previous
import functools

import jax
import jax.numpy as jnp
from jax import lax
from jax.experimental import pallas as pl
from jax.experimental.pallas import tpu as pltpu
from jax.experimental.pallas import tpu_sc as plsc

_EMBED = 32
_BATCH = 16384
_FIELDS = 26
_B = _BATCH * _FIELDS
_NC = 2
_NS = 16
_NW = _NC * _NS
_BPW = _B // _NW
_CHUNK = 1664
_NCHUNK = _BPW // _CHUNK

_mesh = plsc.VectorSubcoreMesh(core_axis_name="c", subcore_axis_name="s")


@functools.partial(
    pl.kernel,
    mesh=_mesh,
    out_type=jax.ShapeDtypeStruct((_B, _EMBED), jnp.float32),
    scratch_types=[
        pltpu.VMEM((_NCHUNK, _CHUNK), jnp.int32),
        pltpu.VMEM((2, _CHUNK, _EMBED), jnp.float32),
        pltpu.SemaphoreType.DMA,
        pltpu.SemaphoreType.DMA,
    ],
    compiler_params=pltpu.CompilerParams(use_tc_tiling_on_sc=False),
)
def _gather(idx_hbm, table_hbm, out_hbm, idx_v, rows_v, sem0, sem1):
    wid = lax.axis_index("s") * _NC + lax.axis_index("c")
    base = wid * _BPW
    sems = (sem0, sem1)
    pltpu.sync_copy(idx_hbm.at[pl.ds(wid * _NCHUNK, _NCHUNK)], idx_v)
    pltpu.async_copy(table_hbm.at[idx_v.at[0]], rows_v.at[0], sems[0])
    for i in range(_NCHUNK):
        if i + 1 < _NCHUNK:
            pltpu.async_copy(
                table_hbm.at[idx_v.at[i + 1]], rows_v.at[(i + 1) % 2],
                sems[(i + 1) % 2])
        pltpu.make_async_copy(
            table_hbm.at[idx_v.at[i]], rows_v.at[i % 2], sems[i % 2]).wait()
        pltpu.sync_copy(rows_v.at[i % 2],
                        out_hbm.at[pl.ds(base + i * _CHUNK, _CHUNK)])


_SLAB_W = 128 * _EMBED
_NSLAB = _B // 128
_SPW = _NSLAB // _NW
_GRP = 4
_GPW = _SPW // _GRP
_GRP_W = _GRP * _SLAB_W


@functools.partial(
    pl.kernel,
    mesh=_mesh,
    out_type=jax.ShapeDtypeStruct((_FIELDS, _EMBED, _BATCH), jnp.float32),
    scratch_types=[
        pltpu.VMEM((_GRP_W,), jnp.float32),
        pltpu.VMEM((_GRP_W,), jnp.float32),
        pltpu.VMEM((1, _EMBED, _GRP * 128 + 9), jnp.float32),
        pltpu.VMEM((1, _EMBED, _GRP * 128 + 9), jnp.float32),
        pltpu.SemaphoreType.DMA,
        pltpu.SemaphoreType.DMA,
        pltpu.SemaphoreType.DMA,
        pltpu.SemaphoreType.DMA,
    ],
    compiler_params=pltpu.CompilerParams(
        use_tc_tiling_on_sc=True, needs_layout_passes=False),
)
def _format_out(lin_hbm, out_hbm, in_v0, in_v1, tr_v0, tr_v1,
                isem0, isem1, osem0, osem1):
    wid = lax.axis_index("s") * _NC + lax.axis_index("c")
    s0 = wid * _SPW
    in_bufs = (in_v0, in_v1)
    tr_bufs = (tr_v0, tr_v1)
    isems = (isem0, isem1)
    osems = (osem0, osem1)

    def out_dst(g):
        s = s0 + g * _GRP
        return out_hbm.at[pl.ds(s // 128, 1), :,
                          pl.ds((s % 128) * 128, _GRP * 128)]

    def tr_src(tr_v):
        return tr_v.at[:, :, pl.ds(0, _GRP * 128)]

    pltpu.async_copy(lin_hbm.at[pl.ds(s0 * _SLAB_W, _GRP_W)],
                     in_v0, isems[0])
    pltpu.async_copy(lin_hbm.at[pl.ds((s0 + _GRP) * _SLAB_W, _GRP_W)],
                     in_v1, isems[1])
    zero16 = lax.iota(jnp.int32, 16) * 0
    evecs = [lax.iota(jnp.int32, 16) + 16 * h for h in range(2)]

    def body(i, carry):
        for b in range(2):
            g = 2 * i + b
            in_v = in_bufs[b]
            tr_v = tr_bufs[b]
            pltpu.make_async_copy(
                lin_hbm.at[pl.ds((s0 + g * _GRP) * _SLAB_W, _GRP_W)],
                in_v, isems[b]).wait()

            @pl.when(g >= 2)
            def _():
                pltpu.make_async_copy(
                    tr_src(tr_v), out_dst(g - 2), osems[b]).wait()

            def rows(rb, carry2):
                for k in range(8):
                    r = rb * 8 + k
                    colv = lax.broadcast(r, (16,))
                    for h in range(2):
                        v = in_v[pl.ds(r * _EMBED + 16 * h, 16)]
                        plsc.store_scatter(
                            tr_v, [zero16, evecs[h], colv], v)
                return carry2

            lax.fori_loop(0, _GRP * 128 // 8, rows, 0)

            pltpu.async_copy(tr_src(tr_v), out_dst(g), osems[b])

            @pl.when(g + 2 < _GPW)
            def _():
                pltpu.async_copy(
                    lin_hbm.at[pl.ds((s0 + (g + 2) * _GRP) * _SLAB_W,
                                     _GRP_W)],
                    in_v, isems[b])
        return carry

    lax.fori_loop(0, _GPW // 2, body, 0)
    pltpu.make_async_copy(tr_src(tr_v0), out_dst(_GPW - 2), osems[0]).wait()
    pltpu.make_async_copy(tr_src(tr_v1), out_dst(_GPW - 1), osems[1]).wait()


def kernel(x, table):
    idx = x.T.reshape(_B // _CHUNK, _CHUNK).astype(jnp.int32)
    lin = _gather(idx, table)
    out_t = _format_out(lin.reshape(-1))
    return jnp.transpose(out_t, (2, 0, 1))
